# baseline (speedup 1.0000x reference)
import jax
import jax.numpy as jnp
from jax.experimental import pallas as pl
from jax.experimental.pallas import tpu as pltpu

C = 128
G = 8
CG = 16
EPS = 1e-5
WP = 24
BF = jnp.bfloat16


def _round8(n):
    return ((n + 7) // 8) * 8


def _make_layer_kernel(D, W, Pp, rows, use_gn, cout):
    D2 = D + 2
    base = WP

    def kern(x_ref, w_ref, b_ref, g_ref, be_ref, o_ref,
             xp_ref, xc_ref, xm_ref, yp_ref, yc_ref, ym_ref):
        o_ref[...] = jnp.zeros_like(o_ref)

        def roll_body(p, _):
            plane = x_ref[0, 0, p]
            pp = jnp.roll(plane, 1, axis=0)
            pm = jnp.roll(plane, -1, axis=0)
            xp_ref[p] = pp.astype(BF)
            xc_ref[p] = plane.astype(BF)
            xm_ref[p] = pm.astype(BF)
            yp_ref[p] = jnp.roll(pp, -WP, axis=0).astype(BF)
            yc_ref[p] = jnp.roll(plane, -WP, axis=0).astype(BF)
            ym_ref[p] = jnp.roll(pm, -WP, axis=0).astype(BF)
            return 0

        jax.lax.fori_loop(0, D2, roll_body, 0)

        mask = ((jax.lax.broadcasted_iota(jnp.int32, (rows, 1), 0) % WP) < W)
        maskf = mask.astype(jnp.float32)
        bias = b_ref[0]

        def body(d, carry):
            s_c, q_c = carry
            acc = None
            for kd in range(3):
                p = d + kd
                wk = w_ref[0, kd]
                for kh in range(3):
                    for kw, v0, v1 in ((0, xp_ref, yp_ref),
                                       (1, xc_ref, yc_ref),
                                       (2, xm_ref, ym_ref)):
                        if kh == 1:
                            src = v1[p, pl.ds(0, rows), :]
                        else:
                            src = v0[p, pl.ds(kh * WP, rows), :]
                        t = 3 * kh + kw
                        pp = jnp.dot(src, wk[t * C:(t + 1) * C, :],
                                     preferred_element_type=jnp.float32)
                        acc = pp if acc is None else acc + pp
            acc = (acc + bias) * maskf
            o_ref[0, 0, d + 1, pl.ds(base, rows), :] = acc
            s_c = s_c + jnp.sum(acc, axis=0, keepdims=True)
            q_c = q_c + jnp.sum(acc * acc, axis=0, keepdims=True)
            return s_c, q_c

        s_c, q_c = jax.lax.fori_loop(
            0, D, body,
            (jnp.zeros((1, cout), jnp.float32), jnp.zeros((1, cout), jnp.float32)))

        if use_gn:
            gi = jax.lax.broadcasted_iota(jnp.int32, (C, C), 0) // CG
            gj = jax.lax.broadcasted_iota(jnp.int32, (C, C), 1) // CG
            agg = (gi == gj).astype(jnp.float32)
            cnt = float(D * W * W * CG)
            gs = jnp.dot(s_c, agg, preferred_element_type=jnp.float32) / cnt
            gq = jnp.dot(q_c, agg, preferred_element_type=jnp.float32) / cnt
            var = gq - gs * gs
            inv = jax.lax.rsqrt(var + EPS)
            scale = g_ref[0]
            beta = be_ref[0]
            a = inv * scale
            bb = beta - gs * inv * scale

            def body2(d, _):
                y = o_ref[0, 0, d + 1, pl.ds(base, rows), :]
                y = jnp.maximum(y * a + bb, 0.0) * maskf
                o_ref[0, 0, d + 1, pl.ds(base, rows), :] = y
                return 0

            jax.lax.fori_loop(0, D, body2, 0)

    return kern


def _layer_call(x, w, b, g, be, D, W, Pp, rows, use_gn, shared_input, cout):
    B = x.shape[0]
    T = w.shape[0]
    D2 = D + 2
    kern = _make_layer_kernel(D, W, Pp, rows, use_gn, cout)
    if shared_input:
        x_spec = pl.BlockSpec((1, 1, D2, Pp, C), lambda bi, ti: (bi, 0, 0, 0, 0))
    else:
        x_spec = pl.BlockSpec((1, 1, D2, Pp, C), lambda bi, ti: (bi, ti, 0, 0, 0))
    w_spec = pl.BlockSpec((1, 3, 9 * C, cout), lambda bi, ti: (ti, 0, 0, 0))
    v_spec = pl.BlockSpec((1, 1, cout), lambda bi, ti: (ti, 0, 0))
    vC_spec = pl.BlockSpec((1, 1, C), lambda bi, ti: (ti, 0, 0))
    o_spec = pl.BlockSpec((1, 1, D2, Pp, cout), lambda bi, ti: (bi, ti, 0, 0, 0))
    scr = pltpu.VMEM((D2, Pp, C), BF)
    return pl.pallas_call(
        kern,
        grid=(B, T),
        in_specs=[x_spec, w_spec, v_spec, vC_spec, vC_spec],
        out_specs=o_spec,
        out_shape=jax.ShapeDtypeStruct((B, T, D2, Pp, cout), jnp.float32),
        scratch_shapes=[scr, scr, scr, scr, scr, scr],
        compiler_params=pltpu.CompilerParams(
            dimension_semantics=("parallel", "parallel")),
    )(x, w, b, g, be)


def _prep_x(feat, D, W, Pp):
    B = feat.shape[0]
    x = jnp.transpose(feat, (0, 2, 3, 4, 1))
    x = jnp.pad(x, ((0, 0), (1, 1), (1, 1), (0, WP - W), (0, 0)))
    x = x.reshape(B, D + 2, (D + 2) * WP, C)
    return x[:, None]


def _prep_w(w):
    o, i = w.shape[0], w.shape[1]
    wt = jnp.transpose(w, (2, 3, 4, 1, 0))
    return wt.reshape(3, 9 * i, o).astype(BF)


def _run_level(feat, params, D):
    W = D
    H2 = D + 2
    Pp = H2 * WP
    rows = _round8((D - 1) * WP + W)
    B = feat.shape[0]

    x = _prep_x(feat, D, W, Pp)
    pc, pr = params['cls'], params['reg']
    for l in range(4):
        wc, bc, gc, bec = pc['conv'][l]
        wr, br, gr, ber = pr['conv'][l]
        w = jnp.stack([_prep_w(wc), _prep_w(wr)])
        b = jnp.stack([bc, br])[:, None, :]
        g = jnp.stack([gc, gr])[:, None, :]
        be = jnp.stack([bec, ber])[:, None, :]
        x = _layer_call(x, w, b, g, be, D, W, Pp, rows, True, l == 0, C)

    (wco, boc), (wro, bor) = pc['out'], pr['out']
    nco, nro = wco.shape[0], wro.shape[0]
    co = 32
    wo = jnp.stack([
        jnp.pad(_prep_w(wco), ((0, 0), (0, 0), (0, co - nco))),
        jnp.pad(_prep_w(wro), ((0, 0), (0, 0), (0, co - nro))),
    ])
    bo = jnp.stack([jnp.pad(boc, (0, co - nco)), jnp.pad(bor, (0, co - nro))])
    bo = bo[:, None, :]
    dummy = jnp.zeros((2, 1, C), jnp.float32)
    o = _layer_call(x, wo, bo, dummy, dummy, D, W, Pp, rows, False, False, co)

    o = o[:, :, 1:D + 1, :, :].reshape(B, 2, D, H2, WP, co)
    o = o[:, :, :, 1:W + 1, :W, :]
    cls = jnp.transpose(o[:, 0, :, :, :, :nco], (0, 4, 1, 2, 3))
    reg = jnp.transpose(o[:, 1, :, :, :, :nro], (0, 4, 1, 2, 3))
    return cls, reg


def kernel(feat0, feat1, params):
    cls0, reg0 = _run_level(feat0, params, 20)
    cls1, reg1 = _run_level(feat1, params, 10)
    return (cls0, cls1, reg0, reg1)


# speedup vs baseline: 1.0482x; 1.0482x over previous
import jax
import jax.numpy as jnp
from jax.experimental import pallas as pl
from jax.experimental.pallas import tpu as pltpu

C = 128
G = 8
CG = 16
EPS = 1e-5
WP = 24
BF = jnp.bfloat16


def _round8(n):
    return ((n + 7) // 8) * 8


def _make_layer_kernel(D, W, Pp, rows, use_gn, cout):
    D2 = D + 2
    base = WP

    def kern(x_ref, w_ref, b_ref, g_ref, be_ref, o_ref, im_ref):
        o_ref[...] = jnp.zeros_like(o_ref)

        def roll_body(p, _):
            plane = x_ref[0, 0, p]
            pp = jnp.roll(plane, 1, axis=0)
            pm = jnp.roll(plane, -1, axis=0)
            for kh in range(3):
                off = kh * WP
                im_ref[p, :, 0 * C + kh * 3 * C:1 * C + kh * 3 * C] = \
                    jax.lax.slice_in_dim(pp, off, off + rows, axis=0).astype(BF)
                im_ref[p, :, 1 * C + kh * 3 * C:2 * C + kh * 3 * C] = \
                    jax.lax.slice_in_dim(plane, off, off + rows, axis=0).astype(BF)
                im_ref[p, :, 2 * C + kh * 3 * C:3 * C + kh * 3 * C] = \
                    jax.lax.slice_in_dim(pm, off, off + rows, axis=0).astype(BF)
            return 0

        jax.lax.fori_loop(0, D2, roll_body, 0)

        mask = ((jax.lax.broadcasted_iota(jnp.int32, (rows, 1), 0) % WP) < W)
        maskf = mask.astype(jnp.float32)
        bias = b_ref[0]

        def body(d, carry):
            s_c, q_c = carry
            acc = None
            for kd in range(3):
                pp = jnp.dot(im_ref[d + kd], w_ref[0, kd],
                             preferred_element_type=jnp.float32)
                acc = pp if acc is None else acc + pp
            acc = (acc + bias) * maskf
            o_ref[0, 0, d + 1, pl.ds(base, rows), :] = acc
            s_c = s_c + jnp.sum(acc, axis=0, keepdims=True)
            q_c = q_c + jnp.sum(acc * acc, axis=0, keepdims=True)
            return s_c, q_c

        s_c, q_c = jax.lax.fori_loop(
            0, D, body,
            (jnp.zeros((1, cout), jnp.float32), jnp.zeros((1, cout), jnp.float32)))

        if use_gn:
            gi = jax.lax.broadcasted_iota(jnp.int32, (C, C), 0) // CG
            gj = jax.lax.broadcasted_iota(jnp.int32, (C, C), 1) // CG
            agg = (gi == gj).astype(jnp.float32)
            cnt = float(D * W * W * CG)
            gs = jnp.dot(s_c, agg, preferred_element_type=jnp.float32) / cnt
            gq = jnp.dot(q_c, agg, preferred_element_type=jnp.float32) / cnt
            var = gq - gs * gs
            inv = jax.lax.rsqrt(var + EPS)
            scale = g_ref[0]
            beta = be_ref[0]
            a = inv * scale
            bb = beta - gs * inv * scale

            def body2(d, _):
                y = o_ref[0, 0, d + 1, pl.ds(base, rows), :]
                y = jnp.maximum(y * a + bb, 0.0) * maskf
                o_ref[0, 0, d + 1, pl.ds(base, rows), :] = y
                return 0

            jax.lax.fori_loop(0, D, body2, 0)

    return kern


def _layer_call(x, w, b, g, be, D, W, Pp, rows, use_gn, shared_input, cout):
    B = x.shape[0]
    T = w.shape[0]
    D2 = D + 2
    kern = _make_layer_kernel(D, W, Pp, rows, use_gn, cout)
    if shared_input:
        x_spec = pl.BlockSpec((1, 1, D2, Pp, C), lambda bi, ti: (bi, 0, 0, 0, 0))
    else:
        x_spec = pl.BlockSpec((1, 1, D2, Pp, C), lambda bi, ti: (bi, ti, 0, 0, 0))
    w_spec = pl.BlockSpec((1, 3, 9 * C, cout), lambda bi, ti: (ti, 0, 0, 0))
    v_spec = pl.BlockSpec((1, 1, cout), lambda bi, ti: (ti, 0, 0))
    vC_spec = pl.BlockSpec((1, 1, C), lambda bi, ti: (ti, 0, 0))
    o_spec = pl.BlockSpec((1, 1, D2, Pp, cout), lambda bi, ti: (bi, ti, 0, 0, 0))
    scr = pltpu.VMEM((D2, rows, 9 * C), BF)
    return pl.pallas_call(
        kern,
        grid=(B, T),
        in_specs=[x_spec, w_spec, v_spec, vC_spec, vC_spec],
        out_specs=o_spec,
        out_shape=jax.ShapeDtypeStruct((B, T, D2, Pp, cout), jnp.float32),
        scratch_shapes=[scr],
        compiler_params=pltpu.CompilerParams(
            dimension_semantics=("parallel", "parallel")),
    )(x, w, b, g, be)


def _prep_x(feat, D, W, Pp):
    B = feat.shape[0]
    x = jnp.transpose(feat, (0, 2, 3, 4, 1))
    x = jnp.pad(x, ((0, 0), (1, 1), (1, 1), (0, WP - W), (0, 0)))
    x = x.reshape(B, D + 2, (D + 2) * WP, C)
    return x[:, None]


def _prep_w(w):
    o, i = w.shape[0], w.shape[1]
    wt = jnp.transpose(w, (2, 3, 4, 1, 0))
    return wt.reshape(3, 9 * i, o).astype(BF)


def _run_level(feat, params, D):
    W = D
    H2 = D + 2
    Pp = H2 * WP
    rows = _round8((D - 1) * WP + W)
    B = feat.shape[0]

    x = _prep_x(feat, D, W, Pp)
    pc, pr = params['cls'], params['reg']
    for l in range(4):
        wc, bc, gc, bec = pc['conv'][l]
        wr, br, gr, ber = pr['conv'][l]
        w = jnp.stack([_prep_w(wc), _prep_w(wr)])
        b = jnp.stack([bc, br])[:, None, :]
        g = jnp.stack([gc, gr])[:, None, :]
        be = jnp.stack([bec, ber])[:, None, :]
        x = _layer_call(x, w, b, g, be, D, W, Pp, rows, True, l == 0, C)

    (wco, boc), (wro, bor) = pc['out'], pr['out']
    nco, nro = wco.shape[0], wro.shape[0]
    co = 32
    wo = jnp.stack([
        jnp.pad(_prep_w(wco), ((0, 0), (0, 0), (0, co - nco))),
        jnp.pad(_prep_w(wro), ((0, 0), (0, 0), (0, co - nro))),
    ])
    bo = jnp.stack([jnp.pad(boc, (0, co - nco)), jnp.pad(bor, (0, co - nro))])
    bo = bo[:, None, :]
    dummy = jnp.zeros((2, 1, C), jnp.float32)
    o = _layer_call(x, wo, bo, dummy, dummy, D, W, Pp, rows, False, False, co)

    o = o[:, :, 1:D + 1, :, :].reshape(B, 2, D, H2, WP, co)
    o = o[:, :, :, 1:W + 1, :W, :]
    cls = jnp.transpose(o[:, 0, :, :, :, :nco], (0, 4, 1, 2, 3))
    reg = jnp.transpose(o[:, 1, :, :, :, :nro], (0, 4, 1, 2, 3))
    return cls, reg


def kernel(feat0, feat1, params):
    cls0, reg0 = _run_level(feat0, params, 20)
    cls1, reg1 = _run_level(feat1, params, 10)
    return (cls0, cls1, reg0, reg1)


# speedup vs baseline: 1.1277x; 1.0758x over previous
import jax
import jax.numpy as jnp
from jax.experimental import pallas as pl
from jax.experimental.pallas import tpu as pltpu

C = 128
G = 8
CG = 16
EPS = 1e-5
WP = 24
NL = 4
BF = jnp.bfloat16


def _round8(n):
    return ((n + 7) // 8) * 8


def _make_tower_kernel(D, W, Pp, rows, co):
    D2 = D + 2
    base = WP
    cnt = float(D * W * W * CG)

    def kern(x_ref, w_ref, b_ref, g_ref, be_ref, wo_ref, bo_ref, o_ref,
             im_ref, pb_ref):
        o_ref[...] = jnp.zeros_like(o_ref)
        pb_ref[...] = jnp.zeros_like(pb_ref)

        def im2col_store(p, plane):
            pp = jnp.roll(plane, 1, axis=0)
            pm = jnp.roll(plane, -1, axis=0)
            for kh in range(3):
                off = kh * WP
                for kw, src in ((0, pp), (1, plane), (2, pm)):
                    t = 3 * kh + kw
                    im_ref[p, :, t * C:(t + 1) * C] = \
                        jax.lax.slice_in_dim(src, off, off + rows,
                                             axis=0).astype(BF)

        def seed_body(p, _):
            im2col_store(p, x_ref[0, 0, p].astype(jnp.float32))
            return 0

        jax.lax.fori_loop(0, D2, seed_body, 0)

        mask = ((jax.lax.broadcasted_iota(jnp.int32, (rows, 1), 0) % WP) < W)
        maskf = mask.astype(jnp.float32)

        gi = jax.lax.broadcasted_iota(jnp.int32, (C, C), 0) // CG
        gj = jax.lax.broadcasted_iota(jnp.int32, (C, C), 1) // CG
        agg = (gi == gj).astype(jnp.float32)

        for l in range(NL):
            bias = b_ref[0, l]

            def body(d, carry, l=l):
                s_c, q_c = carry
                acc = None
                for kd in range(3):
                    pp = jnp.dot(im_ref[d + kd], w_ref[0, l, kd],
                                 preferred_element_type=jnp.float32)
                    acc = pp if acc is None else acc + pp
                acc = (acc + bias) * maskf
                pb_ref[d + 1, pl.ds(base, rows), :] = acc
                s_c = s_c + jnp.sum(acc, axis=0, keepdims=True)
                q_c = q_c + jnp.sum(acc * acc, axis=0, keepdims=True)
                return s_c, q_c

            s_c, q_c = jax.lax.fori_loop(
                0, D, body,
                (jnp.zeros((1, C), jnp.float32), jnp.zeros((1, C), jnp.float32)))

            gs = jnp.dot(s_c, agg, preferred_element_type=jnp.float32) / cnt
            gq = jnp.dot(q_c, agg, preferred_element_type=jnp.float32) / cnt
            var = gq - gs * gs
            inv = jax.lax.rsqrt(var + EPS)
            a = inv * g_ref[0, l]
            bb = be_ref[0, l] - gs * inv * g_ref[0, l]

            def body2(p, _):
                y = pb_ref[p, pl.ds(base, rows), :]
                y = jnp.maximum(y * a + bb, 0.0) * maskf
                pb_ref[p, pl.ds(base, rows), :] = y
                im2col_store(p, pb_ref[p])
                return 0

            jax.lax.fori_loop(1, D + 1, body2, 0)

        biaso = bo_ref[0]

        def bodyf(d, _):
            acc = None
            for kd in range(3):
                pp = jnp.dot(im_ref[d + kd], wo_ref[0, kd],
                             preferred_element_type=jnp.float32)
                acc = pp if acc is None else acc + pp
            acc = (acc + biaso) * maskf
            o_ref[0, 0, d + 1, pl.ds(base, rows), :] = acc
            return 0

        jax.lax.fori_loop(0, D, bodyf, 0)

    return kern


def _tower_call(x, w, b, g, be, wo, bo, D, W, Pp, rows, co):
    B = x.shape[0]
    T = w.shape[0]
    D2 = D + 2
    kern = _make_tower_kernel(D, W, Pp, rows, co)
    x_spec = pl.BlockSpec((1, 1, D2, Pp, C), lambda bi, ti: (bi, 0, 0, 0, 0))
    w_spec = pl.BlockSpec((1, NL, 3, 9 * C, C), lambda bi, ti: (ti, 0, 0, 0, 0))
    v_spec = pl.BlockSpec((1, NL, 1, C), lambda bi, ti: (ti, 0, 0, 0))
    wo_spec = pl.BlockSpec((1, 3, 9 * C, co), lambda bi, ti: (ti, 0, 0, 0))
    bo_spec = pl.BlockSpec((1, 1, co), lambda bi, ti: (ti, 0, 0))
    o_spec = pl.BlockSpec((1, 1, D2, Pp, co), lambda bi, ti: (bi, ti, 0, 0, 0))
    return pl.pallas_call(
        kern,
        grid=(B, T),
        in_specs=[x_spec, w_spec, v_spec, v_spec, v_spec, wo_spec, bo_spec],
        out_specs=o_spec,
        out_shape=jax.ShapeDtypeStruct((B, T, D2, Pp, co), jnp.float32),
        scratch_shapes=[pltpu.VMEM((D2, rows, 9 * C), BF),
                        pltpu.VMEM((D2, Pp, C), jnp.float32)],
        compiler_params=pltpu.CompilerParams(
            dimension_semantics=("parallel", "parallel"),
            vmem_limit_bytes=56 * 1024 * 1024),
    )(x, w, b, g, be, wo, bo)


def _prep_x(feat, D, W, Pp):
    B = feat.shape[0]
    x = jnp.transpose(feat, (0, 2, 3, 4, 1))
    x = jnp.pad(x, ((0, 0), (1, 1), (1, 1), (0, WP - W), (0, 0)))
    x = x.reshape(B, D + 2, (D + 2) * WP, C)
    return x[:, None].astype(BF)


def _prep_w(w):
    o, i = w.shape[0], w.shape[1]
    wt = jnp.transpose(w, (2, 3, 4, 1, 0))
    return wt.reshape(3, 9 * i, o).astype(BF)


def _run_level(feat, params, D):
    W = D
    H2 = D + 2
    Pp = H2 * WP
    rows = _round8((D - 1) * WP + W)
    B = feat.shape[0]

    x = _prep_x(feat, D, W, Pp)
    pc, pr = params['cls'], params['reg']
    w = jnp.stack([jnp.stack([_prep_w(p['conv'][l][0]) for l in range(NL)])
                   for p in (pc, pr)])
    b = jnp.stack([jnp.stack([p['conv'][l][1] for l in range(NL)])
                   for p in (pc, pr)])[:, :, None, :]
    g = jnp.stack([jnp.stack([p['conv'][l][2] for l in range(NL)])
                   for p in (pc, pr)])[:, :, None, :]
    be = jnp.stack([jnp.stack([p['conv'][l][3] for l in range(NL)])
                    for p in (pc, pr)])[:, :, None, :]

    (wco, boc), (wro, bor) = pc['out'], pr['out']
    nco, nro = wco.shape[0], wro.shape[0]
    co = 32
    wo = jnp.stack([
        jnp.pad(_prep_w(wco), ((0, 0), (0, 0), (0, co - nco))),
        jnp.pad(_prep_w(wro), ((0, 0), (0, 0), (0, co - nro))),
    ])
    bo = jnp.stack([jnp.pad(boc, (0, co - nco)), jnp.pad(bor, (0, co - nro))])
    bo = bo[:, None, :]

    o = _tower_call(x, w, b, g, be, wo, bo, D, W, Pp, rows, co)

    o = o[:, :, 1:D + 1, :, :].reshape(B, 2, D, H2, WP, co)
    o = o[:, :, :, 1:W + 1, :W, :]
    cls = jnp.transpose(o[:, 0, :, :, :, :nco], (0, 4, 1, 2, 3))
    reg = jnp.transpose(o[:, 1, :, :, :, :nro], (0, 4, 1, 2, 3))
    return cls, reg


def kernel(feat0, feat1, params):
    cls0, reg0 = _run_level(feat0, params, 20)
    cls1, reg1 = _run_level(feat1, params, 10)
    return (cls0, cls1, reg0, reg1)
